# baseline (speedup 1.0000x reference)
import math

import jax
import jax.numpy as jnp
from jax.experimental import pallas as pl
from jax.experimental.pallas import tpu as pltpu

_TEMPERATURE = 0.1
_LN2 = math.log(2.0)
_C = (1.0 / _TEMPERATURE) / _LN2
_NC = 64
_BM = 512


def _cl_kernel(rows_ref, all_ref, lab_col_ref, row_lab_ref, loss_ref, cnt_ref):
    bm = rows_ref.shape[0]
    b = all_ref.shape[0]
    s2 = jax.lax.dot_general(
        rows_ref[...], all_ref[...],
        dimension_numbers=(((1,), (1,)), ((), ())),
        preferred_element_type=jnp.float32,
    )
    labs_col = lab_col_ref[...]
    row_labs = row_lab_ref[...]

    cls = jax.lax.broadcasted_iota(jnp.int32, (b, 128), 1)
    vf = jnp.where((cls == labs_col) | (cls == _NC), 1.0, 0.0).astype(jnp.float32)
    cc = jnp.sum(vf, axis=0, keepdims=True)
    vb = vf.astype(jnp.bfloat16)

    cls_r = jax.lax.broadcasted_iota(jnp.int32, (bm, 128), 1)
    u = jnp.where(cls_r == row_labs, 1.0, 0.0).astype(jnp.float32)

    e = jnp.exp2(s2)
    m1 = jax.lax.dot_general(
        e.astype(jnp.bfloat16), vb,
        dimension_numbers=(((1,), (0,)), ((), ())),
        preferred_element_type=jnp.float32,
    )
    sum_all = m1[:, _NC:_NC + 1]
    sum_same = jnp.sum(u * m1, axis=1, keepdims=True)
    denom = sum_all - sum_same

    t = jnp.log2(denom + e)
    diff = t - s2
    m2 = jax.lax.dot_general(
        diff.astype(jnp.bfloat16), vb,
        dimension_numbers=(((1,), (0,)), ((), ())),
        preferred_element_type=jnp.float32,
    )
    sum_same_diff = jnp.sum(u * m2, axis=1, keepdims=True)

    corr = jnp.log2(denom + jnp.float32(2.0 ** _C)) - jnp.float32(_C)
    loss2 = sum_same_diff - corr
    cnt = jnp.sum(u * cc, axis=1, keepdims=True) - 1.0

    loss_ref[...] = jnp.full((1, 1, 128), jnp.sum(loss2), jnp.float32)
    cnt_ref[...] = jnp.full((1, 1, 128), jnp.sum(cnt), jnp.float32)


def kernel(embeddings, labels):
    b, d = embeddings.shape
    bm = _BM
    g = b // bm
    labs_col = labels.astype(jnp.int32).reshape(b, 1)
    rows_b = (embeddings * jnp.float32(_C)).astype(jnp.bfloat16)
    all_b = embeddings.astype(jnp.bfloat16)
    loss_p, cnt_p = pl.pallas_call(
        _cl_kernel,
        grid=(g,),
        in_specs=[
            pl.BlockSpec((bm, d), lambda i: (i, 0)),
            pl.BlockSpec((b, d), lambda i: (0, 0)),
            pl.BlockSpec((b, 1), lambda i: (0, 0)),
            pl.BlockSpec((bm, 1), lambda i: (i, 0)),
        ],
        out_specs=[
            pl.BlockSpec((1, 1, 128), lambda i: (i, 0, 0)),
            pl.BlockSpec((1, 1, 128), lambda i: (i, 0, 0)),
        ],
        out_shape=[
            jax.ShapeDtypeStruct((g, 1, 128), jnp.float32),
            jax.ShapeDtypeStruct((g, 1, 128), jnp.float32),
        ],
        compiler_params=pltpu.CompilerParams(
            dimension_semantics=("parallel",),
            vmem_limit_bytes=60 * 1024 * 1024,
        ),
    )(rows_b, all_b, labs_col, labs_col)
    loss_sum = jnp.sum(loss_p[:, 0, 0]) * jnp.float32(_LN2)
    num_pos = jnp.sum(cnt_p[:, 0, 0])
    return loss_sum / jnp.maximum(num_pos, 1.0)


# speedup vs baseline: 1.0988x; 1.0988x over previous
import math

import jax
import jax.numpy as jnp
from jax.experimental import pallas as pl
from jax.experimental.pallas import tpu as pltpu

_TEMPERATURE = 0.1
_LN2 = math.log(2.0)
_C = (1.0 / _TEMPERATURE) / _LN2
_NC = 64
_BM = 256


def _cl_kernel(rows_ref, all_ref, lab_col_ref, row_lab_ref, loss_ref, cnt_ref):
    bm = rows_ref.shape[0]
    b = all_ref.shape[0]
    s2 = jax.lax.dot_general(
        rows_ref[...], all_ref[...],
        dimension_numbers=(((1,), (1,)), ((), ())),
        preferred_element_type=jnp.float32,
    )
    labs_col = lab_col_ref[...]
    row_labs = row_lab_ref[...]

    cls = jax.lax.broadcasted_iota(jnp.int32, (b, 128), 1)
    vf = jnp.where((cls == labs_col) | (cls == _NC), 1.0, 0.0).astype(jnp.float32)
    cc = jnp.sum(vf, axis=0, keepdims=True)
    vb = vf.astype(jnp.bfloat16)

    cls_r = jax.lax.broadcasted_iota(jnp.int32, (bm, 128), 1)
    u = jnp.where(cls_r == row_labs, 1.0, 0.0).astype(jnp.float32)

    e = jnp.exp2(s2)
    m1 = jax.lax.dot_general(
        e.astype(jnp.bfloat16), vb,
        dimension_numbers=(((1,), (0,)), ((), ())),
        preferred_element_type=jnp.float32,
    )
    sum_all = m1[:, _NC:_NC + 1]
    sum_same = jnp.sum(u * m1, axis=1, keepdims=True)
    denom = sum_all - sum_same

    t = jnp.log2(denom + e)
    diff = t - s2
    m2 = jax.lax.dot_general(
        diff.astype(jnp.bfloat16), vb,
        dimension_numbers=(((1,), (0,)), ((), ())),
        preferred_element_type=jnp.float32,
    )
    sum_same_diff = jnp.sum(u * m2, axis=1, keepdims=True)

    corr = jnp.log2(denom + jnp.float32(2.0 ** _C)) - jnp.float32(_C)
    loss2 = sum_same_diff - corr
    cnt = jnp.sum(u * cc, axis=1, keepdims=True) - 1.0

    loss_ref[...] = jnp.full((1, 1, 128), jnp.sum(loss2), jnp.float32)
    cnt_ref[...] = jnp.full((1, 1, 128), jnp.sum(cnt), jnp.float32)


def kernel(embeddings, labels):
    b, d = embeddings.shape
    bm = _BM
    g = b // bm
    labs_col = labels.astype(jnp.int32).reshape(b, 1)
    rows_b = (embeddings * jnp.float32(_C)).astype(jnp.bfloat16)
    all_b = embeddings.astype(jnp.bfloat16)
    loss_p, cnt_p = pl.pallas_call(
        _cl_kernel,
        grid=(g,),
        in_specs=[
            pl.BlockSpec((bm, d), lambda i: (i, 0)),
            pl.BlockSpec((b, d), lambda i: (0, 0)),
            pl.BlockSpec((b, 1), lambda i: (0, 0)),
            pl.BlockSpec((bm, 1), lambda i: (i, 0)),
        ],
        out_specs=[
            pl.BlockSpec((1, 1, 128), lambda i: (i, 0, 0)),
            pl.BlockSpec((1, 1, 128), lambda i: (i, 0, 0)),
        ],
        out_shape=[
            jax.ShapeDtypeStruct((g, 1, 128), jnp.float32),
            jax.ShapeDtypeStruct((g, 1, 128), jnp.float32),
        ],
        compiler_params=pltpu.CompilerParams(
            dimension_semantics=("parallel",),
            vmem_limit_bytes=60 * 1024 * 1024,
        ),
    )(rows_b, all_b, labs_col, labs_col)
    loss_sum = jnp.sum(loss_p[:, 0, 0]) * jnp.float32(_LN2)
    num_pos = jnp.sum(cnt_p[:, 0, 0])
    return loss_sum / jnp.maximum(num_pos, 1.0)


# speedup vs baseline: 1.1666x; 1.0617x over previous
import math

import jax
import jax.numpy as jnp
from jax.experimental import pallas as pl
from jax.experimental.pallas import tpu as pltpu

_TEMPERATURE = 0.1
_LN2 = math.log(2.0)
_C = (1.0 / _TEMPERATURE) / _LN2
_NC = 64
_BM = 256
_NK = 4


def _cl_kernel(rows_ref, all_ref, lab_col_ref, row_lab_ref, loss_ref, cnt_ref,
               allb_ref, vb_ref, cc_ref):
    i = pl.program_id(0)
    bm = rows_ref.shape[0]
    b = all_ref.shape[0]
    bk = b // _NK

    @pl.when(i == 0)
    def _init():
        allb_ref[...] = all_ref[...].astype(jnp.bfloat16)
        labs_col = lab_col_ref[...]
        cls = jax.lax.broadcasted_iota(jnp.int32, (b, 128), 1)
        vf = jnp.where((cls == labs_col) | (cls == _NC), 1.0, 0.0)
        cc_ref[...] = jnp.sum(vf, axis=0, keepdims=True)
        vb_ref[...] = vf.astype(jnp.bfloat16)

    rows_b = (rows_ref[...] * jnp.float32(_C)).astype(jnp.bfloat16)
    row_labs = row_lab_ref[...]

    cls_r = jax.lax.broadcasted_iota(jnp.int32, (bm, 128), 1)
    u = jnp.where(cls_r == row_labs, 1.0, 0.0).astype(jnp.float32)

    s2_chunks, eb_chunks = [], []
    m1 = jnp.zeros((bm, 128), jnp.float32)
    for c in range(_NK):
        s2_c = jax.lax.dot_general(
            rows_b, allb_ref[pl.ds(c * bk, bk), :],
            dimension_numbers=(((1,), (1,)), ((), ())),
            preferred_element_type=jnp.float32,
        )
        eb_c = jnp.exp2(s2_c).astype(jnp.bfloat16)
        m1 = m1 + jax.lax.dot_general(
            eb_c, vb_ref[pl.ds(c * bk, bk), :],
            dimension_numbers=(((1,), (0,)), ((), ())),
            preferred_element_type=jnp.float32,
        )
        s2_chunks.append(s2_c)
        eb_chunks.append(eb_c)

    sum_all = m1[:, _NC:_NC + 1]
    sum_same = jnp.sum(u * m1, axis=1, keepdims=True)
    denom = sum_all - sum_same

    m2 = jnp.zeros((bm, 128), jnp.float32)
    for c in range(_NK):
        t_c = jnp.log2(denom + eb_chunks[c].astype(jnp.float32))
        diff_c = t_c - s2_chunks[c]
        m2 = m2 + jax.lax.dot_general(
            diff_c.astype(jnp.bfloat16), vb_ref[pl.ds(c * bk, bk), :],
            dimension_numbers=(((1,), (0,)), ((), ())),
            preferred_element_type=jnp.float32,
        )
    sum_same_diff = jnp.sum(u * m2, axis=1, keepdims=True)

    corr = jnp.log2(denom + jnp.float32(2.0 ** _C)) - jnp.float32(_C)
    loss2 = sum_same_diff - corr
    cnt = jnp.sum(u * cc_ref[...], axis=1, keepdims=True) - 1.0

    loss_ref[...] = jnp.full((1, 1, 128), jnp.sum(loss2), jnp.float32)
    cnt_ref[...] = jnp.full((1, 1, 128), jnp.sum(cnt), jnp.float32)


def kernel(embeddings, labels):
    b, d = embeddings.shape
    bm = _BM
    g = b // bm
    labs_col = labels.astype(jnp.int32).reshape(b, 1)
    loss_p, cnt_p = pl.pallas_call(
        _cl_kernel,
        grid=(g,),
        in_specs=[
            pl.BlockSpec((bm, d), lambda i: (i, 0)),
            pl.BlockSpec((b, d), lambda i: (0, 0)),
            pl.BlockSpec((b, 1), lambda i: (0, 0)),
            pl.BlockSpec((bm, 1), lambda i: (i, 0)),
        ],
        out_specs=[
            pl.BlockSpec((1, 1, 128), lambda i: (i, 0, 0)),
            pl.BlockSpec((1, 1, 128), lambda i: (i, 0, 0)),
        ],
        out_shape=[
            jax.ShapeDtypeStruct((g, 1, 128), jnp.float32),
            jax.ShapeDtypeStruct((g, 1, 128), jnp.float32),
        ],
        scratch_shapes=[
            pltpu.VMEM((b, d), jnp.bfloat16),
            pltpu.VMEM((b, 128), jnp.bfloat16),
            pltpu.VMEM((1, 128), jnp.float32),
        ],
        compiler_params=pltpu.CompilerParams(
            dimension_semantics=("arbitrary",),
            vmem_limit_bytes=60 * 1024 * 1024,
        ),
    )(embeddings, embeddings, labs_col, labs_col)
    loss_sum = jnp.sum(loss_p[:, 0, 0]) * jnp.float32(_LN2)
    num_pos = jnp.sum(cnt_p[:, 0, 0])
    return loss_sum / jnp.maximum(num_pos, 1.0)


# speedup vs baseline: 1.4330x; 1.2284x over previous
import math

import jax
import jax.numpy as jnp
from jax.experimental import pallas as pl
from jax.experimental.pallas import tpu as pltpu

_TEMPERATURE = 0.1
_LN2 = math.log(2.0)
_C = (1.0 / _TEMPERATURE) / _LN2
_NC = 64
_BM = 256
_NK = 4


def _cl_kernel(all_ref, lab_col_ref, loss_ref, cnt_ref):
    b, d = all_ref.shape
    bm = _BM
    nb = b // bm
    bk = b // _NK

    all_f = all_ref[...]
    allb = all_f.astype(jnp.bfloat16)
    allbs = (all_f * jnp.float32(_C)).astype(jnp.bfloat16)
    labs = lab_col_ref[...]

    cls = jax.lax.broadcasted_iota(jnp.int32, (b, 128), 1)
    vf = jnp.where((cls == labs) | (cls == _NC), 1.0, 0.0)
    cc = jnp.sum(vf, axis=0, keepdims=True)
    vb = vf.astype(jnp.bfloat16)
    cls_r = jax.lax.broadcasted_iota(jnp.int32, (bm, 128), 1)

    def phase1(r):
        rows_b = allbs[r * bm:(r + 1) * bm, :]
        s2s, ebs = [], []
        m1 = jnp.zeros((bm, 128), jnp.float32)
        for c in range(_NK):
            s2_c = jax.lax.dot_general(
                rows_b, allb[c * bk:(c + 1) * bk, :],
                dimension_numbers=(((1,), (1,)), ((), ())),
                preferred_element_type=jnp.float32,
            )
            eb_c = jnp.exp2(s2_c).astype(jnp.bfloat16)
            m1 = m1 + jax.lax.dot_general(
                eb_c, vb[c * bk:(c + 1) * bk, :],
                dimension_numbers=(((1,), (0,)), ((), ())),
                preferred_element_type=jnp.float32,
            )
            s2s.append(s2_c)
            ebs.append(eb_c)
        u = jnp.where(cls_r == labs[r * bm:(r + 1) * bm, :], 1.0, 0.0)
        sum_all = m1[:, _NC:_NC + 1]
        sum_same = jnp.sum(u * m1, axis=1, keepdims=True)
        denom = sum_all - sum_same
        return s2s, ebs, denom, u

    def phase2(st):
        s2s, ebs, denom, u = st
        m2 = jnp.zeros((bm, 128), jnp.float32)
        for c in range(_NK):
            t_c = jnp.log2(denom + ebs[c].astype(jnp.float32))
            diff_c = t_c - s2s[c]
            m2 = m2 + jax.lax.dot_general(
                diff_c.astype(jnp.bfloat16), vb[c * bk:(c + 1) * bk, :],
                dimension_numbers=(((1,), (0,)), ((), ())),
                preferred_element_type=jnp.float32,
            )
        sum_same_diff = jnp.sum(u * m2, axis=1, keepdims=True)
        corr = jnp.log2(denom + jnp.float32(2.0 ** _C)) - jnp.float32(_C)
        loss2 = sum_same_diff - corr
        cnt2 = jnp.sum(u * cc, axis=1, keepdims=True) - 1.0
        return jnp.sum(loss2), jnp.sum(cnt2)

    loss_sum = jnp.float32(0.0)
    cnt_sum = jnp.float32(0.0)
    state = phase1(0)
    for r in range(nb):
        nxt = phase1(r + 1) if r + 1 < nb else None
        dl, dc = phase2(state)
        loss_sum = loss_sum + dl
        cnt_sum = cnt_sum + dc
        state = nxt

    loss_ref[...] = jnp.full((1, 1, 128), loss_sum, jnp.float32)
    cnt_ref[...] = jnp.full((1, 1, 128), cnt_sum, jnp.float32)


def kernel(embeddings, labels):
    b, d = embeddings.shape
    labs_col = labels.astype(jnp.int32).reshape(b, 1)
    loss_p, cnt_p = pl.pallas_call(
        _cl_kernel,
        grid=(1,),
        in_specs=[
            pl.BlockSpec((b, d), lambda i: (0, 0)),
            pl.BlockSpec((b, 1), lambda i: (0, 0)),
        ],
        out_specs=[
            pl.BlockSpec((1, 1, 128), lambda i: (0, 0, 0)),
            pl.BlockSpec((1, 1, 128), lambda i: (0, 0, 0)),
        ],
        out_shape=[
            jax.ShapeDtypeStruct((1, 1, 128), jnp.float32),
            jax.ShapeDtypeStruct((1, 1, 128), jnp.float32),
        ],
        compiler_params=pltpu.CompilerParams(
            dimension_semantics=("arbitrary",),
            vmem_limit_bytes=60 * 1024 * 1024,
        ),
    )(embeddings, labs_col)
    loss_sum = loss_p[0, 0, 0] * jnp.float32(_LN2)
    num_pos = cnt_p[0, 0, 0]
    return loss_sum / jnp.maximum(num_pos, 1.0)


# speedup vs baseline: 1.4907x; 1.0402x over previous
import math

import jax
import jax.numpy as jnp
from jax.experimental import pallas as pl
from jax.experimental.pallas import tpu as pltpu

_TEMPERATURE = 0.1
_LN2 = math.log(2.0)
_C = (1.0 / _TEMPERATURE) / _LN2
_NC = 64
_BM = 256
_NK = 4


def _cl_kernel(all_ref, lab_col_ref, loss_ref, cnt_ref):
    b, d = all_ref.shape
    bm = _BM
    nb = b // bm
    bk = b // _NK

    all_f = all_ref[...]
    allb = all_f.astype(jnp.bfloat16)
    allbs = (all_f * jnp.float32(_C)).astype(jnp.bfloat16)
    labs = lab_col_ref[...]
    labs_row = jnp.reshape(labs, (1, b))

    cls = jax.lax.broadcasted_iota(jnp.int32, (b, 128), 1)
    vf = jnp.where(cls == labs, 1.0, 0.0)
    cc = jnp.sum(vf, axis=0, keepdims=True)
    cls_r = jax.lax.broadcasted_iota(jnp.int32, (bm, 128), 1)

    def phase1(r):
        rows_b = allbs[r * bm:(r + 1) * bm, :]
        row_labs = labs[r * bm:(r + 1) * bm, :]
        s2s, es = [], []
        sum_same_e = jnp.zeros((bm, 1), jnp.float32)
        sum_all_e = jnp.zeros((bm, 1), jnp.float32)
        for c in range(_NK):
            s2_c = jax.lax.dot_general(
                rows_b, allb[c * bk:(c + 1) * bk, :],
                dimension_numbers=(((1,), (1,)), ((), ())),
                preferred_element_type=jnp.float32,
            )
            e_c = jnp.exp2(s2_c)
            same_c = row_labs == labs_row[:, c * bk:(c + 1) * bk]
            sum_all_e = sum_all_e + jnp.sum(e_c, axis=1, keepdims=True)
            sum_same_e = sum_same_e + jnp.sum(
                jnp.where(same_c, e_c, 0.0), axis=1, keepdims=True)
            s2s.append(s2_c)
            es.append(e_c)
        u = jnp.where(cls_r == row_labs, 1.0, 0.0)
        denom = sum_all_e - sum_same_e
        return r, s2s, es, denom, u

    def phase2(st):
        r, s2s, es, denom, u = st
        row_labs = labs[r * bm:(r + 1) * bm, :]
        loss_acc = jnp.zeros((bm, 1), jnp.float32)
        for c in range(_NK):
            t_c = jnp.log2(denom + es[c])
            diff_c = t_c - s2s[c]
            same_c = row_labs == labs_row[:, c * bk:(c + 1) * bk]
            loss_acc = loss_acc + jnp.sum(
                jnp.where(same_c, diff_c, 0.0), axis=1, keepdims=True)
        corr = jnp.log2(denom + jnp.float32(2.0 ** _C)) - jnp.float32(_C)
        loss2 = loss_acc - corr
        cnt2 = jnp.sum(u * cc, axis=1, keepdims=True) - 1.0
        return jnp.sum(loss2), jnp.sum(cnt2)

    loss_sum = jnp.float32(0.0)
    cnt_sum = jnp.float32(0.0)
    state = phase1(0)
    for r in range(nb):
        nxt = phase1(r + 1) if r + 1 < nb else None
        dl, dc = phase2(state)
        loss_sum = loss_sum + dl
        cnt_sum = cnt_sum + dc
        state = nxt

    loss_ref[...] = jnp.full((1, 1, 128), loss_sum, jnp.float32)
    cnt_ref[...] = jnp.full((1, 1, 128), cnt_sum, jnp.float32)


def kernel(embeddings, labels):
    b, d = embeddings.shape
    labs_col = labels.astype(jnp.int32).reshape(b, 1)
    loss_p, cnt_p = pl.pallas_call(
        _cl_kernel,
        grid=(1,),
        in_specs=[
            pl.BlockSpec((b, d), lambda i: (0, 0)),
            pl.BlockSpec((b, 1), lambda i: (0, 0)),
        ],
        out_specs=[
            pl.BlockSpec((1, 1, 128), lambda i: (0, 0, 0)),
            pl.BlockSpec((1, 1, 128), lambda i: (0, 0, 0)),
        ],
        out_shape=[
            jax.ShapeDtypeStruct((1, 1, 128), jnp.float32),
            jax.ShapeDtypeStruct((1, 1, 128), jnp.float32),
        ],
        compiler_params=pltpu.CompilerParams(
            dimension_semantics=("arbitrary",),
            vmem_limit_bytes=60 * 1024 * 1024,
        ),
    )(embeddings, labs_col)
    loss_sum = loss_p[0, 0, 0] * jnp.float32(_LN2)
    num_pos = cnt_p[0, 0, 0]
    return loss_sum / jnp.maximum(num_pos, 1.0)


# speedup vs baseline: 1.7273x; 1.1587x over previous
import math

import jax
import jax.numpy as jnp
from jax.experimental import pallas as pl
from jax.experimental.pallas import tpu as pltpu

_TEMPERATURE = 0.1
_LN2 = math.log(2.0)
_C = (1.0 / _TEMPERATURE) / _LN2
_NC = 64
_BM = 256
_NK = 4


def _cl_kernel(all_ref, lab_col_ref, loss_ref, cnt_ref):
    b, d = all_ref.shape
    bm = _BM
    nb = b // bm
    bk = b // _NK

    all_f = all_ref[...]
    allb = all_f.astype(jnp.bfloat16)
    allbs = (all_f * jnp.float32(_C)).astype(jnp.bfloat16)
    labs = lab_col_ref[...]
    labs_row = jnp.reshape(labs, (1, b))

    cls = jax.lax.broadcasted_iota(jnp.int32, (b, 128), 1)
    vf = jnp.where(cls == labs, 1.0, 0.0)
    cc = jnp.sum(vf, axis=0, keepdims=True)
    cls_r = jax.lax.broadcasted_iota(jnp.int32, (bm, 128), 1)

    def p1_chunk(r, c, rows_b, row_labs):
        s2_c = jax.lax.dot_general(
            rows_b, allb[c * bk:(c + 1) * bk, :],
            dimension_numbers=(((1,), (1,)), ((), ())),
            preferred_element_type=jnp.float32,
        )
        e_c = jnp.exp2(s2_c)
        same_c = row_labs == labs_row[:, c * bk:(c + 1) * bk]
        dneg = jnp.sum(jnp.where(same_c, 0.0, e_c), axis=1, keepdims=True)
        return s2_c, e_c, dneg

    def p2_chunk(r, c, s2_c, e_c, denom, row_labs):
        t_c = jnp.log2(denom + e_c)
        diff_c = t_c - s2_c
        same_c = row_labs == labs_row[:, c * bk:(c + 1) * bk]
        return jnp.sum(jnp.where(same_c, diff_c, 0.0), axis=1, keepdims=True)

    def block_inputs(r):
        return (allbs[r * bm:(r + 1) * bm, :], labs[r * bm:(r + 1) * bm, :])

    loss_sum = jnp.float32(0.0)
    cnt_sum = jnp.float32(0.0)

    rows_b, row_labs = block_inputs(0)
    s2s, es, dnegs = zip(*[p1_chunk(0, c, rows_b, row_labs) for c in range(_NK)])
    state = (0, list(s2s), list(es), sum(dnegs), row_labs)
    for r in range(nb):
        pr, s2s_p, es_p, denom, labs_p = state
        nxt = None
        if r + 1 < nb:
            rows_b, row_labs = block_inputs(r + 1)
            n_s2, n_e, n_dneg = [], [], []
        loss_acc = jnp.zeros((bm, 1), jnp.float32)
        for c in range(_NK):
            if r + 1 < nb:
                s2_c, e_c, dneg_c = p1_chunk(r + 1, c, rows_b, row_labs)
                n_s2.append(s2_c)
                n_e.append(e_c)
                n_dneg.append(dneg_c)
            loss_acc = loss_acc + p2_chunk(pr, c, s2s_p[c], es_p[c], denom, labs_p)
        if r + 1 < nb:
            state = (r + 1, n_s2, n_e, sum(n_dneg), row_labs)
        corr = jnp.log2(denom + jnp.float32(2.0 ** _C)) - jnp.float32(_C)
        u = jnp.where(cls_r == labs_p, 1.0, 0.0)
        cnt2 = jnp.sum(u * cc, axis=1, keepdims=True) - 1.0
        loss_sum = loss_sum + jnp.sum(loss_acc - corr)
        cnt_sum = cnt_sum + jnp.sum(cnt2)

    loss_ref[...] = jnp.full((1, 1, 128), loss_sum, jnp.float32)
    cnt_ref[...] = jnp.full((1, 1, 128), cnt_sum, jnp.float32)


def kernel(embeddings, labels):
    b, d = embeddings.shape
    labs_col = labels.astype(jnp.int32).reshape(b, 1)
    loss_p, cnt_p = pl.pallas_call(
        _cl_kernel,
        grid=(1,),
        in_specs=[
            pl.BlockSpec((b, d), lambda i: (0, 0)),
            pl.BlockSpec((b, 1), lambda i: (0, 0)),
        ],
        out_specs=[
            pl.BlockSpec((1, 1, 128), lambda i: (0, 0, 0)),
            pl.BlockSpec((1, 1, 128), lambda i: (0, 0, 0)),
        ],
        out_shape=[
            jax.ShapeDtypeStruct((1, 1, 128), jnp.float32),
            jax.ShapeDtypeStruct((1, 1, 128), jnp.float32),
        ],
        compiler_params=pltpu.CompilerParams(
            dimension_semantics=("arbitrary",),
            vmem_limit_bytes=60 * 1024 * 1024,
        ),
    )(embeddings, labs_col)
    loss_sum = loss_p[0, 0, 0] * jnp.float32(_LN2)
    num_pos = cnt_p[0, 0, 0]
    return loss_sum / jnp.maximum(num_pos, 1.0)
